# baseline (speedup 1.0000x reference)
import functools

import jax
import jax.numpy as jnp
from jax import lax
from jax.experimental import pallas as pl
from jax.experimental.pallas import tpu as pltpu
from jax.experimental.pallas import tpu_sc as plsc

_TR = 16


def _pick_chunk(n: int, cap: int = 5120) -> int:
    for c in range(min(n, cap), _TR - 1, -1):
        if n % c == 0 and c % _TR == 0:
            return c
    return n


def _topk_body(n_chunk, n_total, x_ref, w_ref, db_ref, val_ref, idx_ref,
               qn_ref, s_ref, accv_ref, acci_ref):
    i = pl.program_id(0)

    @pl.when(i == 0)
    def _init():
        feats = jnp.dot(x_ref[...], w_ref[...],
                        preferred_element_type=jnp.float32)
        qnorm = jnp.sqrt(jnp.sum(feats * feats, axis=1, keepdims=True))
        qn_ref[...] = feats / (qnorm + 1e-8)
        accv_ref[...] = jnp.full(accv_ref.shape, -jnp.inf, jnp.float32)
        acci_ref[...] = jnp.zeros(acci_ref.shape, jnp.int32)

    db = db_ref[...]
    dnorm = jnp.sqrt(jnp.sum(db * db, axis=1, keepdims=True))
    dn = db / (dnorm + 1e-8)
    s_ref[...] = lax.dot_general(dn, qn_ref[...], (((1,), (1,)), ((), ())),
                                 preferred_element_type=jnp.float32)

    n_tiles = n_chunk // _TR
    tile0 = i * n_tiles

    def tile_step(t, carry):
        av, ai = carry
        blk = s_ref[pl.ds(t * _TR, _TR), :]
        a, b = blk[:8], blk[8:]
        pm = jnp.maximum(a, b)
        gt2 = 2 * (tile0 + t)
        e = jnp.where(b > a, gt2 + 1, gt2)
        cmp = pm > av
        return jnp.maximum(av, pm), jnp.where(cmp, e, ai)

    av, ai = lax.fori_loop(0, n_tiles, tile_step,
                           (accv_ref[...], acci_ref[...]), unroll=2)
    accv_ref[...] = av
    acci_ref[...] = ai

    @pl.when(i == pl.num_programs(0) - 1)
    def _finish():
        v = accv_ref[...]
        e = acci_ref[...]
        cls = lax.broadcasted_iota(jnp.int32, v.shape, 0)
        row = (e >> 1) * _TR + (e & 1) * 8 + cls
        m = jnp.max(v, axis=0)
        cand = jnp.min(jnp.where(v == m[None, :], row, n_total), axis=0)
        val_ref[...] = m
        idx_ref[...] = cand


def _topk_call(x, W, db):
    b, d = x.shape
    n = db.shape[0]
    chunk = _pick_chunk(n)
    nsteps = n // chunk
    return pl.pallas_call(
        functools.partial(_topk_body, chunk, n),
        grid=(nsteps,),
        in_specs=[
            pl.BlockSpec((b, d), lambda i: (0, 0)),
            pl.BlockSpec((d, d), lambda i: (0, 0)),
            pl.BlockSpec((chunk, d), lambda i: (i, 0)),
        ],
        out_specs=[
            pl.BlockSpec((b,), lambda i: (0,)),
            pl.BlockSpec((b,), lambda i: (0,)),
        ],
        out_shape=[
            jax.ShapeDtypeStruct((b,), jnp.float32),
            jax.ShapeDtypeStruct((b,), jnp.int32),
        ],
        scratch_shapes=[
            pltpu.VMEM((b, d), jnp.float32),
            pltpu.VMEM((chunk, b), jnp.float32),
            pltpu.VMEM((8, b), jnp.float32),
            pltpu.VMEM((8, b), jnp.int32),
        ],
    )(x, W, db)


def _label_gather(labels, idx):
    b = idx.shape[0]
    info = plsc.get_sparse_core_info()
    nw = info.num_cores * info.num_subcores
    bpw = b // nw
    mesh = plsc.VectorSubcoreMesh(core_axis_name="c", subcore_axis_name="s")

    @functools.partial(
        pl.kernel,
        mesh=mesh,
        out_type=jax.ShapeDtypeStruct((b,), jnp.int32),
        scratch_types=[
            pltpu.VMEM((bpw,), jnp.int32),
            pltpu.VMEM((bpw,), jnp.int32),
            pltpu.SemaphoreType.DMA,
        ],
    )
    def k(labels_hbm, idx_hbm, out_hbm, idx_v, vals_v, sem):
        wid = lax.axis_index("s") * info.num_cores + lax.axis_index("c")
        base = wid * bpw
        pltpu.sync_copy(idx_hbm.at[pl.ds(base, bpw)], idx_v)
        pltpu.async_copy(labels_hbm.at[idx_v], vals_v, sem).wait()
        pltpu.sync_copy(vals_v, out_hbm.at[pl.ds(base, bpw)])

    return k(labels, idx)


def kernel(x, W, db_features, db_labels):
    top_vals, top_idx = _topk_call(x, W, db_features)
    pred = _label_gather(db_labels, top_idx)
    return top_vals, top_idx, pred


# speedup vs baseline: 1.1361x; 1.1361x over previous
import functools

import jax
import jax.numpy as jnp
from jax import lax
from jax.experimental import pallas as pl
from jax.experimental.pallas import tpu as pltpu
from jax.experimental.pallas import tpu_sc as plsc

_TR = 16


def _pick_chunk(n: int, cap: int = 5120) -> int:
    for c in range(min(n, cap), _TR - 1, -1):
        if n % c == 0 and c % _TR == 0:
            return c
    return n


def _topk_body(n_chunk, n_total, x_ref, w_ref, db_ref, val_ref, idx_ref,
               qn_ref, s_ref, accv_ref, acci_ref):
    i = pl.program_id(0)

    @pl.when(i == 0)
    def _init():
        feats = jnp.dot(x_ref[...], w_ref[...],
                        preferred_element_type=jnp.float32)
        qnorm = jnp.sqrt(jnp.sum(feats * feats, axis=1, keepdims=True))
        qn_ref[...] = feats / (qnorm + 1e-8)
        accv_ref[...] = jnp.full(accv_ref.shape, -jnp.inf, jnp.float32)
        acci_ref[...] = jnp.zeros(acci_ref.shape, jnp.int32)

    db = db_ref[...]
    dnorm = jnp.sqrt(jnp.sum(db * db, axis=1, keepdims=True))
    dn = db / (dnorm + 1e-8)
    s_ref[...] = lax.dot_general(dn, qn_ref[...], (((1,), (1,)), ((), ())),
                                 preferred_element_type=jnp.float32)

    n_tiles = n_chunk // _TR
    tile0 = i * n_tiles

    def tile_step(t, carry):
        av, ai = carry
        blk = s_ref[pl.ds(t * _TR, _TR), :]
        a, b = blk[:8], blk[8:]
        pm = jnp.maximum(a, b)
        gt2 = 2 * (tile0 + t)
        e = jnp.where(b > a, gt2 + 1, gt2)
        cmp = pm > av
        return jnp.maximum(av, pm), jnp.where(cmp, e, ai)

    av, ai = lax.fori_loop(0, n_tiles, tile_step,
                           (accv_ref[...], acci_ref[...]), unroll=10)
    accv_ref[...] = av
    acci_ref[...] = ai

    @pl.when(i == pl.num_programs(0) - 1)
    def _finish():
        v = accv_ref[...]
        e = acci_ref[...]
        cls = lax.broadcasted_iota(jnp.int32, v.shape, 0)
        row = (e >> 1) * _TR + (e & 1) * 8 + cls
        m = jnp.max(v, axis=0)
        cand = jnp.min(jnp.where(v == m[None, :], row, n_total), axis=0)
        val_ref[...] = m
        idx_ref[...] = cand


def _topk_call(x, W, db):
    b, d = x.shape
    n = db.shape[0]
    chunk = _pick_chunk(n)
    nsteps = n // chunk
    return pl.pallas_call(
        functools.partial(_topk_body, chunk, n),
        grid=(nsteps,),
        in_specs=[
            pl.BlockSpec((b, d), lambda i: (0, 0)),
            pl.BlockSpec((d, d), lambda i: (0, 0)),
            pl.BlockSpec((chunk, d), lambda i: (i, 0)),
        ],
        out_specs=[
            pl.BlockSpec((b,), lambda i: (0,)),
            pl.BlockSpec((b,), lambda i: (0,)),
        ],
        out_shape=[
            jax.ShapeDtypeStruct((b,), jnp.float32),
            jax.ShapeDtypeStruct((b,), jnp.int32),
        ],
        scratch_shapes=[
            pltpu.VMEM((b, d), jnp.float32),
            pltpu.VMEM((chunk, b), jnp.float32),
            pltpu.VMEM((8, b), jnp.float32),
            pltpu.VMEM((8, b), jnp.int32),
        ],
    )(x, W, db)


def _label_gather(labels, idx):
    b = idx.shape[0]
    info = plsc.get_sparse_core_info()
    nw = info.num_cores * info.num_subcores
    bpw = b // nw
    mesh = plsc.VectorSubcoreMesh(core_axis_name="c", subcore_axis_name="s")

    @functools.partial(
        pl.kernel,
        mesh=mesh,
        out_type=jax.ShapeDtypeStruct((b,), jnp.int32),
        scratch_types=[
            pltpu.VMEM((bpw,), jnp.int32),
            pltpu.VMEM((bpw,), jnp.int32),
            pltpu.SemaphoreType.DMA,
        ],
    )
    def k(labels_hbm, idx_hbm, out_hbm, idx_v, vals_v, sem):
        wid = lax.axis_index("s") * info.num_cores + lax.axis_index("c")
        base = wid * bpw
        pltpu.sync_copy(idx_hbm.at[pl.ds(base, bpw)], idx_v)
        pltpu.async_copy(labels_hbm.at[idx_v], vals_v, sem).wait()
        pltpu.sync_copy(vals_v, out_hbm.at[pl.ds(base, bpw)])

    return k(labels, idx)


def kernel(x, W, db_features, db_labels):
    top_vals, top_idx = _topk_call(x, W, db_features)
    pred = _label_gather(db_labels, top_idx)
    return top_vals, top_idx, pred


# speedup vs baseline: 1.3829x; 1.2173x over previous
import functools

import jax
import jax.numpy as jnp
from jax import lax
from jax.experimental import pallas as pl
from jax.experimental.pallas import tpu as pltpu
from jax.experimental.pallas import tpu_sc as plsc


def _pick_chunk(n: int, cap: int = 5120) -> int:
    for c in range(min(n, cap), 7, -1):
        if n % c == 0 and c % 8 == 0:
            return c
    return n


def _topk_body(n_chunk, x_ref, w_ref, db_ref, val_ref, idx_ref, qn_ref):
    i = pl.program_id(0)

    @pl.when(i == 0)
    def _init():
        feats = jnp.dot(x_ref[...], w_ref[...],
                        preferred_element_type=jnp.float32)
        qnorm = jnp.sqrt(jnp.sum(feats * feats, axis=1, keepdims=True))
        qn_ref[...] = feats / (qnorm + 1e-8)
        val_ref[...] = jnp.full(val_ref.shape, -jnp.inf, jnp.float32)
        idx_ref[...] = jnp.zeros(idx_ref.shape, jnp.int32)

    db = db_ref[...]
    dnorm = jnp.sqrt(jnp.sum(db * db, axis=1, keepdims=True))
    dn = db / (dnorm + 1e-8)
    s = lax.dot_general(dn, qn_ref[...], (((1,), (1,)), ((), ())),
                        preferred_element_type=jnp.float32)
    m = jnp.max(s, axis=0)
    cand = jnp.argmax(s, axis=0).astype(jnp.int32)
    gidx = cand + i * n_chunk
    better = m > val_ref[...]
    val_ref[...] = jnp.where(better, m, val_ref[...])
    idx_ref[...] = jnp.where(better, gidx, idx_ref[...])


def _topk_call(x, W, db):
    b, d = x.shape
    n = db.shape[0]
    chunk = _pick_chunk(n)
    nsteps = n // chunk
    return pl.pallas_call(
        functools.partial(_topk_body, chunk),
        grid=(nsteps,),
        in_specs=[
            pl.BlockSpec((b, d), lambda i: (0, 0)),
            pl.BlockSpec((d, d), lambda i: (0, 0)),
            pl.BlockSpec((chunk, d), lambda i: (i, 0)),
        ],
        out_specs=[
            pl.BlockSpec((b,), lambda i: (0,)),
            pl.BlockSpec((b,), lambda i: (0,)),
        ],
        out_shape=[
            jax.ShapeDtypeStruct((b,), jnp.float32),
            jax.ShapeDtypeStruct((b,), jnp.int32),
        ],
        scratch_shapes=[pltpu.VMEM((b, d), jnp.float32)],
    )(x, W, db)


def _label_gather(labels, idx):
    b = idx.shape[0]
    info = plsc.get_sparse_core_info()
    nw = info.num_cores * info.num_subcores
    bpw = b // nw
    mesh = plsc.VectorSubcoreMesh(core_axis_name="c", subcore_axis_name="s")

    @functools.partial(
        pl.kernel,
        mesh=mesh,
        out_type=jax.ShapeDtypeStruct((b,), jnp.int32),
        scratch_types=[
            pltpu.VMEM((bpw,), jnp.int32),
            pltpu.VMEM((bpw,), jnp.int32),
            pltpu.SemaphoreType.DMA,
        ],
    )
    def k(labels_hbm, idx_hbm, out_hbm, idx_v, vals_v, sem):
        wid = lax.axis_index("s") * info.num_cores + lax.axis_index("c")
        base = wid * bpw
        pltpu.sync_copy(idx_hbm.at[pl.ds(base, bpw)], idx_v)
        pltpu.async_copy(labels_hbm.at[idx_v], vals_v, sem).wait()
        pltpu.sync_copy(vals_v, out_hbm.at[pl.ds(base, bpw)])

    return k(labels, idx)


def kernel(x, W, db_features, db_labels):
    top_vals, top_idx = _topk_call(x, W, db_features)
    pred = _label_gather(db_labels, top_idx)
    return top_vals, top_idx, pred
